# baseline (speedup 1.0000x reference)
import functools

import jax
import jax.numpy as jnp
import numpy as np
from jax.experimental import pallas as pl
from jax.experimental.pallas import tpu as pltpu

_TB = 128


def _band_matrices(w, j_in, j_real, j_out):
    k, _, cin, cout = w.shape
    e = np.zeros((5, j_in, j_out), np.float32)
    for kw in range(5):
        for jp in range(j_out):
            j = 2 * jp + kw - 1
            if 0 <= j < j_real:
                e[kw, j, jp] = 1.0
    t = jnp.einsum("wjp,hwco->hjcpo", jnp.asarray(e), w)
    return t.reshape(5, j_in * cin, j_out * cout).astype(jnp.bfloat16)


def _l1_body(x_ref, w_ref, s_ref, t_ref, o_ref, *, tb):
    f = x_ref[...].astype(jnp.bfloat16)
    zpad = jnp.zeros((tb, 512), jnp.bfloat16)
    cols = [zpad]
    for i in range(15):
        acc = None
        for c in range(3):
            if i == 0:
                lhs = f[:, c * 1024:c * 1024 + 128]
                wm = w_ref[c, 32:160]
            else:
                st = c * 1024 + (2 * i - 1) * 32
                lhs = f[:, st:st + 160]
                wm = w_ref[c]
            d = jnp.dot(lhs, wm, preferred_element_type=jnp.float32)
            acc = d if acc is None else acc + d
        y = acc * s_ref[...] + t_ref[...]
        cols.append(jnp.maximum(y, 0.0).astype(jnp.bfloat16))
    cols.append(zpad)
    o_ref[...] = jnp.concatenate(cols, axis=-1)


def _strip_body(a_ref, w_ref, s_ref, t_ref, o_ref, *, tb, ho):
    a = a_ref[...]
    zpad = jnp.zeros((tb, 512), jnp.bfloat16)
    cols = [zpad]
    for i in range(ho):
        d = jnp.dot(a[:, 1024 * i:1024 * i + 2560], w_ref[...],
                    preferred_element_type=jnp.float32)
        y = d * s_ref[...] + t_ref[...]
        cols.append(jnp.maximum(y, 0.0).astype(jnp.bfloat16))
    cols.append(zpad)
    o_ref[...] = jnp.concatenate(cols, axis=-1)


def _l34_body(a_ref, w3_ref, s3_ref, t3_ref, w4_ref, s4_ref, t4_ref,
              z_ref, rf_ref, *, tb):
    a = a_ref[...]
    zpad = jnp.zeros((tb, 512), jnp.bfloat16)
    cols = [zpad]
    for i in range(3):
        d = jnp.dot(a[:, 1024 * i:1024 * i + 2560], w3_ref[...],
                    preferred_element_type=jnp.float32)
        y = d * s3_ref[...] + t3_ref[...]
        cols.append(jnp.maximum(y, 0.0).astype(jnp.bfloat16))
    cols.append(zpad)
    lhs4 = jnp.concatenate(cols, axis=-1)
    acc = jnp.dot(lhs4, w4_ref[...], preferred_element_type=jnp.float32)
    yout = acc * s4_ref[...] + t4_ref[...]
    z_ref[...] = yout[:, :100]
    rf_ref[...] = 1.0 / (1.0 + jnp.exp(-yout[:, 100:101]))


def kernel(x, w0, b0, scale0, shift0, w1, b1, scale1, shift1,
           w2, b2, scale2, shift2, w3, b3, scale3, shift3):
    B = x.shape[0]
    tb = min(_TB, B)
    grid = (B // tb,)
    par = pltpu.CompilerParams(dimension_semantics=("parallel",))
    full = lambda shape: pl.BlockSpec(shape, lambda i: (0,) * len(shape))

    def affine(b, s, t, j_out, cout):
        se = jnp.tile(s.reshape(1, cout), (1, j_out))
        te = jnp.tile((b * s + t).reshape(1, cout), (1, j_out))
        pad = 512 - j_out * cout
        return (jnp.pad(se, ((0, 0), (0, pad))).astype(jnp.float32),
                jnp.pad(te, ((0, 0), (0, pad))).astype(jnp.float32))

    e1 = np.zeros((5, 32, 15), np.float32)
    for kw in range(5):
        for jp in range(15):
            j = 2 * jp + kw - 1
            if 0 <= j < 32:
                e1[kw, j, jp] = 1.0
    t1m = jnp.einsum("kjp,hkco->chjpo", jnp.asarray(e1), w0)
    wh1 = jnp.pad(t1m.reshape(3, 160, 480),
                  ((0, 0), (0, 0), (0, 32))).astype(jnp.bfloat16)
    wh2 = jnp.pad(_band_matrices(w1, 16, 15, 7),
                  ((0, 0), (0, 0), (0, 64))).reshape(2560, 512)
    wh3 = jnp.pad(_band_matrices(w2, 8, 7, 3),
                  ((0, 0), (0, 0), (0, 128))).reshape(2560, 512)
    wh4 = jnp.pad(w3[:, 1:4], ((0, 0), (0, 1), (0, 0), (0, 0)))
    wh4 = wh4.reshape(2560, 101).astype(jnp.bfloat16)

    s1, t1 = affine(b0, scale0, shift0, 15, 32)
    s2, t2 = affine(b1, scale1, shift1, 7, 64)
    s3, t3 = affine(b2, scale2, shift2, 3, 128)
    s4 = scale3.reshape(1, 101).astype(jnp.float32)
    t4 = (b3 * scale3 + shift3).reshape(1, 101).astype(jnp.float32)

    y1 = pl.pallas_call(
        functools.partial(_l1_body, tb=tb),
        grid=grid,
        in_specs=[pl.BlockSpec((tb, 3072), lambda i: (i, 0)),
                  full((3, 160, 512)), full((1, 512)), full((1, 512))],
        out_shape=jax.ShapeDtypeStruct((B, 17 * 512), jnp.bfloat16),
        out_specs=pl.BlockSpec((tb, 17 * 512), lambda i: (i, 0)),
        compiler_params=par,
    )(x.reshape(B, 3072), wh1, s1, t1)

    y2 = pl.pallas_call(
        functools.partial(_strip_body, tb=tb, ho=7),
        grid=grid,
        in_specs=[pl.BlockSpec((tb, 17 * 512), lambda i: (i, 0)),
                  full((2560, 512)), full((1, 512)), full((1, 512))],
        out_shape=jax.ShapeDtypeStruct((B, 9 * 512), jnp.bfloat16),
        out_specs=pl.BlockSpec((tb, 9 * 512), lambda i: (i, 0)),
        compiler_params=par,
    )(y1, wh2, s2, t2)

    z, rf = pl.pallas_call(
        functools.partial(_l34_body, tb=tb),
        grid=grid,
        in_specs=[pl.BlockSpec((tb, 9 * 512), lambda i: (i, 0)),
                  full((2560, 512)), full((1, 512)), full((1, 512)),
                  full((2560, 101)), full((1, 101)), full((1, 101))],
        out_shape=(jax.ShapeDtypeStruct((B, 100), jnp.float32),
                   jax.ShapeDtypeStruct((B, 1), jnp.float32)),
        out_specs=(pl.BlockSpec((tb, 100), lambda i: (i, 0)),
                   pl.BlockSpec((tb, 1), lambda i: (i, 0))),
        compiler_params=par,
    )(y2, wh3, s3, t3, wh4, s4, t4)
    return z, rf[:, 0]


# speedup vs baseline: 1.0967x; 1.0967x over previous
import functools

import jax
import jax.numpy as jnp
import numpy as np
from jax.experimental import pallas as pl
from jax.experimental.pallas import tpu as pltpu

_TB = 256


def _band_matrices(w, j_in, j_real, j_out):
    k, _, cin, cout = w.shape
    e = np.zeros((5, j_in, j_out), np.float32)
    for kw in range(5):
        for jp in range(j_out):
            j = 2 * jp + kw - 1
            if 0 <= j < j_real:
                e[kw, j, jp] = 1.0
    t = jnp.einsum("wjp,hwco->hjcpo", jnp.asarray(e), w)
    return t.reshape(5, j_in * cin, j_out * cout).astype(jnp.bfloat16)


def _fused_body(x_ref, w1_ref, s1_ref, t1_ref, w2_ref, s2_ref, t2_ref,
                w3_ref, s3_ref, t3_ref, w4_ref, s4_ref, t4_ref,
                z_ref, rf_ref, *, tb):
    f = x_ref[...].astype(jnp.bfloat16)
    zpad = jnp.zeros((tb, 512), jnp.bfloat16)

    rows = [zpad]
    for i in range(15):
        acc = None
        for c in range(3):
            if i == 0:
                lhs = f[:, c * 1024:c * 1024 + 128]
                wm = w1_ref[c, 32:160]
            else:
                st = c * 1024 + (2 * i - 1) * 32
                lhs = f[:, st:st + 160]
                wm = w1_ref[c]
            d = jnp.dot(lhs, wm, preferred_element_type=jnp.float32)
            acc = d if acc is None else acc + d
        y = acc * s1_ref[...] + t1_ref[...]
        rows.append(jnp.maximum(y, 0.0).astype(jnp.bfloat16))
    rows.append(zpad)

    for wb_ref, s_ref, t_ref, ho in ((w2_ref, s2_ref, t2_ref, 7),
                                     (w3_ref, s3_ref, t3_ref, 3)):
        nxt = [zpad]
        for i in range(ho):
            lhs = jnp.concatenate(rows[2 * i:2 * i + 5], axis=-1)
            d = jnp.dot(lhs, wb_ref[...], preferred_element_type=jnp.float32)
            y = d * s_ref[...] + t_ref[...]
            nxt.append(jnp.maximum(y, 0.0).astype(jnp.bfloat16))
        nxt.append(zpad)
        rows = nxt

    lhs4 = jnp.concatenate(rows, axis=-1)
    acc = jnp.dot(lhs4, w4_ref[...], preferred_element_type=jnp.float32)
    yout = acc * s4_ref[...] + t4_ref[...]
    z_ref[...] = yout[:, :100]
    rf_ref[...] = 1.0 / (1.0 + jnp.exp(-yout[:, 100:101]))


def kernel(x, w0, b0, scale0, shift0, w1, b1, scale1, shift1,
           w2, b2, scale2, shift2, w3, b3, scale3, shift3):
    B = x.shape[0]
    tb = min(_TB, B)

    def affine(b, s, t, j_out, cout):
        se = jnp.tile(s.reshape(1, cout), (1, j_out))
        te = jnp.tile((b * s + t).reshape(1, cout), (1, j_out))
        pad = 512 - j_out * cout
        return (jnp.pad(se, ((0, 0), (0, pad))).astype(jnp.float32),
                jnp.pad(te, ((0, 0), (0, pad))).astype(jnp.float32))

    e1 = np.zeros((5, 32, 15), np.float32)
    for kw in range(5):
        for jp in range(15):
            j = 2 * jp + kw - 1
            if 0 <= j < 32:
                e1[kw, j, jp] = 1.0
    t1m = jnp.einsum("kjp,hkco->chjpo", jnp.asarray(e1), w0)
    wh1 = jnp.pad(t1m.reshape(3, 160, 480),
                  ((0, 0), (0, 0), (0, 32))).astype(jnp.bfloat16)
    wh2 = jnp.pad(_band_matrices(w1, 16, 15, 7),
                  ((0, 0), (0, 0), (0, 64))).reshape(2560, 512)
    wh3 = jnp.pad(_band_matrices(w2, 8, 7, 3),
                  ((0, 0), (0, 0), (0, 128))).reshape(2560, 512)
    wh4 = jnp.pad(w3[:, 1:4], ((0, 0), (0, 1), (0, 0), (0, 0)))
    wh4 = wh4.reshape(2560, 101).astype(jnp.bfloat16)

    s1, t1 = affine(b0, scale0, shift0, 15, 32)
    s2, t2 = affine(b1, scale1, shift1, 7, 64)
    s3, t3 = affine(b2, scale2, shift2, 3, 128)
    s4 = scale3.reshape(1, 101).astype(jnp.float32)
    t4 = (b3 * scale3 + shift3).reshape(1, 101).astype(jnp.float32)

    full = lambda shape: pl.BlockSpec(shape, lambda i: (0,) * len(shape))
    z, rf = pl.pallas_call(
        functools.partial(_fused_body, tb=tb),
        grid=(B // tb,),
        in_specs=[
            pl.BlockSpec((tb, 3072), lambda i: (i, 0)),
            full((3, 160, 512)), full((1, 512)), full((1, 512)),
            full((2560, 512)), full((1, 512)), full((1, 512)),
            full((2560, 512)), full((1, 512)), full((1, 512)),
            full((2560, 101)), full((1, 101)), full((1, 101)),
        ],
        out_shape=(jax.ShapeDtypeStruct((B, 100), jnp.float32),
                   jax.ShapeDtypeStruct((B, 1), jnp.float32)),
        out_specs=(pl.BlockSpec((tb, 100), lambda i: (i, 0)),
                   pl.BlockSpec((tb, 1), lambda i: (i, 0))),
        compiler_params=pltpu.CompilerParams(
            dimension_semantics=("parallel",)),
    )(x.reshape(B, 3072), wh1, s1, t1, wh2, s2, t2, wh3, s3, t3,
      wh4, s4, t4)
    return z, rf[:, 0]
